# baseline (speedup 1.0000x reference)
import functools

import jax
import jax.numpy as jnp
from jax import lax
from jax.experimental import pallas as pl
from jax.experimental.pallas import tpu as pltpu
from jax.experimental.pallas import tpu_sc as plsc

NNODE = 10000
NEDGE = 320000
FIN = 128
FH = 64
NGRAPH = 64
NCLS = 10

NC = 2
NS = 16
NW = NC * NS

CH = 128
NCHUNK = 80
EPAD = NW * NCHUNK * CH
RPT = 632
NPAD = NS * RPT
DW = 16
KBUF = 4

PKSHIFT = 14
HS_RPT = 632


def _sc_mesh():
    return plsc.VectorSubcoreMesh(
        core_axis_name="c", subcore_axis_name="s", num_cores=NC, num_subcores=NS
    )


@functools.cache
def _deg_sc_built():
    return functools.partial(
        pl.kernel,
        out_type=jax.ShapeDtypeStruct((NC, NPAD, DW), jnp.float32),
        mesh=_sc_mesh(),
        scratch_types=[
            pltpu.VMEM((NCHUNK, CH), jnp.int32),
            pltpu.VMEM((CH, DW), jnp.float32),
            pltpu.VMEM_SHARED((NPAD, DW), jnp.float32),
            pltpu.SemaphoreType.DMA,
        ],
        compiler_params=pltpu.CompilerParams(use_tc_tiling_on_sc=False),
    )(_deg_sc_body)


def _deg_sc_body(dstr_hbm, ones_hbm, zeros16_hbm, out_hbm, dst_v, ones_v,
                 acc_sh, sem):
    c = lax.axis_index("c")
    s = lax.axis_index("s")
    wid = c * NS + s
    pltpu.sync_copy(dstr_hbm.at[pl.ds(wid * NCHUNK, NCHUNK)], dst_v)
    pltpu.sync_copy(ones_hbm, ones_v)
    pltpu.sync_copy(zeros16_hbm.at[pl.ds(s * RPT, RPT)], acc_sh.at[pl.ds(s * RPT, RPT)])
    plsc.subcore_barrier()

    def body(t, carry):
        base = t * KBUF
        for b in range(KBUF):
            pltpu.async_copy(ones_v, acc_sh.at[dst_v.at[base + b]], sem,
                             add=True)
        for b in range(KBUF):
            pltpu.make_async_copy(ones_v, acc_sh.at[dst_v.at[base + b]],
                                  sem).wait()
        return carry

    lax.fori_loop(0, NCHUNK // KBUF, body, 0)
    plsc.subcore_barrier()
    pltpu.sync_copy(acc_sh.at[pl.ds(s * RPT, RPT)], out_hbm.at[c, pl.ds(s * RPT, RPT)])


@functools.cache
def _agg_sc_built():
    return functools.partial(
        pl.kernel,
        out_type=jax.ShapeDtypeStruct((NC, NPAD, FH), jnp.float32),
        mesh=_sc_mesh(),
        scratch_types=[
            pltpu.VMEM((NCHUNK, CH), jnp.int32),
            pltpu.VMEM((KBUF, CH), jnp.int32),
            pltpu.VMEM((KBUF, CH), jnp.int32),
            [pltpu.VMEM((CH, FH), jnp.float32) for _ in range(KBUF)],
            [pltpu.SemaphoreType.DMA for _ in range(KBUF)],
            [pltpu.SemaphoreType.DMA for _ in range(KBUF)],
            pltpu.VMEM_SHARED((NPAD, FH), jnp.float32),
            pltpu.VMEM_SHARED((NNODE, FH), jnp.float32),
        ],
        compiler_params=pltpu.CompilerParams(use_tc_tiling_on_sc=False),
    )(_agg_sc_body)


def _agg_sc_body(hs_hbm, pkr_hbm, out_hbm,
                 pk_v, src_v, dst_v, rows, gsem, ssem,
                 acc_sh, hs_sh):
    c = lax.axis_index("c")
    s = lax.axis_index("s")
    wid = c * NS + s
    pltpu.sync_copy(pkr_hbm.at[pl.ds(wid * NCHUNK, NCHUNK)], pk_v)

    @pl.when(s < NS - 1)
    def _():
        pltpu.sync_copy(hs_hbm.at[pl.ds(s * HS_RPT, HS_RPT)],
                        hs_sh.at[pl.ds(s * HS_RPT, HS_RPT)])

    @pl.when(s == NS - 1)
    def _():
        pltpu.sync_copy(
            hs_hbm.at[pl.ds((NS - 1) * HS_RPT, NNODE - (NS - 1) * HS_RPT)],
            hs_sh.at[pl.ds((NS - 1) * HS_RPT, NNODE - (NS - 1) * HS_RPT)])

    def _zrow(i, carry):
        for j in range(FH // 16):
            rows[0][i, pl.ds(j * 16, 16)] = jnp.zeros((16,), jnp.float32)
        return carry

    lax.fori_loop(0, CH, _zrow, 0)
    for k in range(RPT // CH):
        pltpu.sync_copy(rows[0], acc_sh.at[pl.ds(s * RPT + k * CH, CH)])
    rem = RPT % CH
    if rem:
        pltpu.sync_copy(rows[0].at[pl.ds(0, rem)],
                        acc_sh.at[pl.ds(s * RPT + (RPT // CH) * CH, rem)])

    def _unpack_src(chunk, b):
        for j in range(CH // 16):
            v = pk_v[chunk, pl.ds(j * 16, 16)]
            src_v[b, pl.ds(j * 16, 16)] = lax.shift_right_logical(v, PKSHIFT)

    def _unpack_dst(chunk, b):
        for j in range(CH // 16):
            v = pk_v[chunk, pl.ds(j * 16, 16)]
            dst_v[b, pl.ds(j * 16, 16)] = v & ((1 << PKSHIFT) - 1)

    for b in range(KBUF):
        _unpack_src(b, b)
        _unpack_dst(b, b)

    plsc.subcore_barrier()

    for b in range(KBUF):
        pltpu.async_copy(hs_sh.at[src_v.at[b]], rows[b], gsem[b])

    def body(t, carry):
        nbase = (t + 1) * KBUF
        for b in range(KBUF):
            pltpu.make_async_copy(
                hs_sh.at[src_v.at[b]], rows[b], gsem[b]).wait()
            pltpu.async_copy(rows[b], acc_sh.at[dst_v.at[b]], ssem[b],
                             add=True)
        for b in range(KBUF):
            _unpack_src(nbase + b, b)
        for b in range(KBUF):
            pltpu.make_async_copy(
                rows[b], acc_sh.at[dst_v.at[b]], ssem[b]).wait()
            _unpack_dst(nbase + b, b)
            pltpu.async_copy(hs_sh.at[src_v.at[b]], rows[b], gsem[b])
        return carry

    lax.fori_loop(0, NCHUNK // KBUF - 1, body, 0)
    for b in range(KBUF):
        pltpu.make_async_copy(
            hs_sh.at[src_v.at[b]], rows[b], gsem[b]).wait()
        pltpu.async_copy(rows[b], acc_sh.at[dst_v.at[b]], ssem[b],
                         add=True)
    for b in range(KBUF):
        pltpu.make_async_copy(
            rows[b], acc_sh.at[dst_v.at[b]], ssem[b]).wait()
    plsc.subcore_barrier()
    pltpu.sync_copy(acc_sh.at[pl.ds(s * RPT, RPT)],
                    out_hbm.at[c, pl.ds(s * RPT, RPT)])


def _tc_first(x_ref, w_ref, degp_ref, hs_ref, dis_ref):
    deg = degp_ref[0] + degp_ref[1] + 1.0
    dis = 1.0 / jnp.sqrt(deg)
    h = jnp.dot(x_ref[...], w_ref[...], preferred_element_type=jnp.float32)
    hs_ref[...] = h * dis
    dis_ref[...] = dis


def _acc_rows(accp_ref):
    return accp_ref[0] + accp_ref[1]


def _tc_mid(accp_ref, hsp_ref, dis_ref, b_ref, w_ref, hs_ref):
    dis = dis_ref[...]
    xk = dis * (_acc_rows(accp_ref) + hsp_ref[...]) + b_ref[...]
    xk = jnp.maximum(xk, 0.0)
    h = jnp.dot(xk, w_ref[...], preferred_element_type=jnp.float32)
    hs_ref[...] = h * dis


def _tc_final(accp_ref, hsp_ref, dis_ref, b_ref, batch_r_ref, batch_c_ref,
              gate_w_ref, gate_b_ref, lin_w_ref, lin_b_ref, out_ref):
    dis = dis_ref[...]
    x4 = dis * (_acc_rows(accp_ref) + hsp_ref[...]) + b_ref[...]
    x4 = jnp.maximum(x4, 0.0)
    gate = jnp.dot(x4, gate_w_ref[...],
                   preferred_element_type=jnp.float32) + gate_b_ref[...]
    gids_r = lax.broadcasted_iota(jnp.int32, (NNODE, NGRAPH), 1)
    oh = gids_r == batch_r_ref[...]
    masked = jnp.where(oh, gate, -1e30)
    m = jnp.max(masked, axis=0, keepdims=True)
    m = jnp.where(m > -1e29, m, 0.0)
    mb = jnp.sum(jnp.where(oh, m, 0.0), axis=1, keepdims=True)
    e = jnp.exp(gate - mb)
    ohf = oh.astype(jnp.float32)
    denom = jnp.sum(ohf * e, axis=0, keepdims=True)
    denom_b = jnp.sum(jnp.where(oh, denom, 0.0), axis=1, keepdims=True)
    alpha = e / jnp.maximum(denom_b, 1e-16)
    gids_c = lax.broadcasted_iota(jnp.int32, (NGRAPH, NNODE), 0)
    oht = (gids_c == batch_c_ref[...]).astype(jnp.float32)
    pooled = jnp.dot(oht, alpha * x4, preferred_element_type=jnp.float32)
    out_ref[...] = jnp.dot(pooled, lin_w_ref[...],
                           preferred_element_type=jnp.float32) + lin_b_ref[...]


def _tc_call(body, out_shape, *args):
    return pl.pallas_call(body, out_shape=out_shape)(*args)


def kernel(x, edge_index, batch, W0, b0, W1, b1, W2, b2, W3, b3,
           gate_w, gate_b, lin_w, lin_b):
    f32 = jnp.float32
    src = edge_index[0]
    dst = edge_index[1]
    pad = EPAD - NEDGE
    srcp = jnp.concatenate([src, jnp.zeros((pad,), jnp.int32)])
    dstp = jnp.concatenate([dst, jnp.full((pad,), NNODE, jnp.int32)])
    dstr = dstp.reshape(NW * NCHUNK, CH)
    pkr = ((srcp << PKSHIFT) | dstp).reshape(NW * NCHUNK, CH)
    ones16 = jnp.ones((CH, DW), f32)
    zeros16 = jnp.zeros((NPAD, DW), f32)

    degp = _deg_sc_built()(dstr, ones16, zeros16)
    degp2 = degp[:, :NNODE, :1]

    hs, dis = _tc_call(
        _tc_first,
        (jax.ShapeDtypeStruct((NNODE, FH), f32),
         jax.ShapeDtypeStruct((NNODE, 1), f32)),
        x, W0, degp2)

    for b, W in ((b0, W1), (b1, W2), (b2, W3)):
        accp = _agg_sc_built()(hs, pkr)
        hs = _tc_call(
            _tc_mid,
            jax.ShapeDtypeStruct((NNODE, FH), f32),
            accp[:, :NNODE, :], hs, dis, b.reshape(1, FH), W)

    accp = _agg_sc_built()(hs, pkr)
    out = _tc_call(
        _tc_final,
        jax.ShapeDtypeStruct((NGRAPH, NCLS), f32),
        accp[:, :NNODE, :], hs, dis, b3.reshape(1, FH),
        batch.reshape(NNODE, 1), batch.reshape(1, NNODE),
        gate_w, gate_b.reshape(1, 1), lin_w, lin_b.reshape(1, NCLS))
    return out


# speedup vs baseline: 1.0134x; 1.0134x over previous
import functools

import jax
import jax.numpy as jnp
from jax import lax
from jax.experimental import pallas as pl
from jax.experimental.pallas import tpu as pltpu
from jax.experimental.pallas import tpu_sc as plsc

NNODE = 10000
NEDGE = 320000
FIN = 128
FH = 64
NGRAPH = 64
NCLS = 10

NC = 2
NS = 16
NW = NC * NS

CH = 128
NCHUNK = 80
EPAD = NW * NCHUNK * CH
RPT = 632
NPAD = NS * RPT
DW = 16
KBUF = 4

PKSHIFT = 14
HS_RPT = 632


def _sc_mesh():
    return plsc.VectorSubcoreMesh(
        core_axis_name="c", subcore_axis_name="s", num_cores=NC, num_subcores=NS
    )


@functools.cache
def _deg_sc_built():
    return functools.partial(
        pl.kernel,
        out_type=jax.ShapeDtypeStruct((NC, NPAD, DW), jnp.float32),
        mesh=_sc_mesh(),
        scratch_types=[
            pltpu.VMEM((NCHUNK, CH), jnp.int32),
            pltpu.VMEM((CH, DW), jnp.float32),
            pltpu.VMEM_SHARED((NPAD, DW), jnp.float32),
            pltpu.SemaphoreType.DMA,
        ],
        compiler_params=pltpu.CompilerParams(use_tc_tiling_on_sc=False),
    )(_deg_sc_body)


def _deg_sc_body(dstr_hbm, ones_hbm, zeros16_hbm, out_hbm, dst_v, ones_v,
                 acc_sh, sem):
    c = lax.axis_index("c")
    s = lax.axis_index("s")
    wid = c * NS + s
    pltpu.sync_copy(dstr_hbm.at[pl.ds(wid * NCHUNK, NCHUNK)], dst_v)
    pltpu.sync_copy(ones_hbm, ones_v)
    pltpu.sync_copy(zeros16_hbm.at[pl.ds(s * RPT, RPT)], acc_sh.at[pl.ds(s * RPT, RPT)])
    plsc.subcore_barrier()

    def body(t, carry):
        base = t * KBUF
        for b in range(KBUF):
            pltpu.async_copy(ones_v, acc_sh.at[dst_v.at[base + b]], sem,
                             add=True)
        for b in range(KBUF):
            pltpu.make_async_copy(ones_v, acc_sh.at[dst_v.at[base + b]],
                                  sem).wait()
        return carry

    lax.fori_loop(0, NCHUNK // KBUF, body, 0)
    plsc.subcore_barrier()
    pltpu.sync_copy(acc_sh.at[pl.ds(s * RPT, RPT)], out_hbm.at[c, pl.ds(s * RPT, RPT)])


@functools.cache
def _agg_sc_built():
    return functools.partial(
        pl.kernel,
        out_type=jax.ShapeDtypeStruct((NC, NPAD, FH), jnp.float32),
        mesh=_sc_mesh(),
        scratch_types=[
            pltpu.VMEM((NCHUNK, CH), jnp.int32),
            pltpu.VMEM((2 * KBUF, CH), jnp.int32),
            pltpu.VMEM((2 * KBUF, CH), jnp.int32),
            [pltpu.VMEM((CH, FH), jnp.float32) for _ in range(KBUF)],
            [pltpu.SemaphoreType.DMA for _ in range(KBUF)],
            [pltpu.SemaphoreType.DMA for _ in range(KBUF)],
            pltpu.VMEM_SHARED((NPAD, FH), jnp.float32),
            pltpu.VMEM_SHARED((NNODE, FH), jnp.float32),
        ],
        compiler_params=pltpu.CompilerParams(use_tc_tiling_on_sc=False),
    )(_agg_sc_body)


def _agg_sc_body(hs_hbm, pkr_hbm, out_hbm,
                 pk_v, src_v, dst_v, rows, gsem, ssem,
                 acc_sh, hs_sh):
    c = lax.axis_index("c")
    s = lax.axis_index("s")
    wid = c * NS + s
    pltpu.sync_copy(pkr_hbm.at[pl.ds(wid * NCHUNK, NCHUNK)], pk_v)

    @pl.when(s < NS - 1)
    def _():
        pltpu.sync_copy(hs_hbm.at[pl.ds(s * HS_RPT, HS_RPT)],
                        hs_sh.at[pl.ds(s * HS_RPT, HS_RPT)])

    @pl.when(s == NS - 1)
    def _():
        pltpu.sync_copy(
            hs_hbm.at[pl.ds((NS - 1) * HS_RPT, NNODE - (NS - 1) * HS_RPT)],
            hs_sh.at[pl.ds((NS - 1) * HS_RPT, NNODE - (NS - 1) * HS_RPT)])

    def _zrow(i, carry):
        for j in range(FH // 16):
            rows[0][i, pl.ds(j * 16, 16)] = jnp.zeros((16,), jnp.float32)
        return carry

    lax.fori_loop(0, CH, _zrow, 0)
    for k in range(RPT // CH):
        pltpu.sync_copy(rows[0], acc_sh.at[pl.ds(s * RPT + k * CH, CH)])
    rem = RPT % CH
    if rem:
        pltpu.sync_copy(rows[0].at[pl.ds(0, rem)],
                        acc_sh.at[pl.ds(s * RPT + (RPT // CH) * CH, rem)])

    def _unpack(chunk, slot):
        for j in range(CH // 16):
            v = pk_v[chunk, pl.ds(j * 16, 16)]
            src_v[slot, pl.ds(j * 16, 16)] = lax.shift_right_logical(
                v, PKSHIFT)
            dst_v[slot, pl.ds(j * 16, 16)] = v & ((1 << PKSHIFT) - 1)

    for b in range(KBUF):
        _unpack(b, b)

    plsc.subcore_barrier()

    nblk = NCHUNK // KBUF

    def body(t, carry):
        par = (t % 2) * KBUF
        npar = KBUF - par
        for b in range(KBUF):
            pltpu.async_copy(hs_sh.at[src_v.at[par + b]], rows[b], gsem[b])
        @pl.when(t + 1 < nblk)
        def _():
            for b in range(KBUF):
                _unpack((t + 1) * KBUF + b, npar + b)

        for b in range(KBUF):
            pltpu.make_async_copy(
                hs_sh.at[src_v.at[par + b]], rows[b], gsem[b]).wait()
            pltpu.async_copy(rows[b], acc_sh.at[dst_v.at[par + b]],
                             ssem[b], add=True)
        for b in range(KBUF):
            pltpu.make_async_copy(
                rows[b], acc_sh.at[dst_v.at[par + b]], ssem[b]).wait()
        return carry

    lax.fori_loop(0, nblk, body, 0)
    plsc.subcore_barrier()
    pltpu.sync_copy(acc_sh.at[pl.ds(s * RPT, RPT)],
                    out_hbm.at[c, pl.ds(s * RPT, RPT)])


def _tc_first(x_ref, w_ref, degp_ref, hs_ref, dis_ref):
    deg = degp_ref[0] + degp_ref[1] + 1.0
    dis = 1.0 / jnp.sqrt(deg)
    h = jnp.dot(x_ref[...], w_ref[...], preferred_element_type=jnp.float32)
    hs_ref[...] = h * dis
    dis_ref[...] = dis


def _acc_rows(accp_ref):
    return accp_ref[0] + accp_ref[1]


def _tc_mid(accp_ref, hsp_ref, dis_ref, b_ref, w_ref, hs_ref):
    dis = dis_ref[...]
    xk = dis * (_acc_rows(accp_ref) + hsp_ref[...]) + b_ref[...]
    xk = jnp.maximum(xk, 0.0)
    h = jnp.dot(xk, w_ref[...], preferred_element_type=jnp.float32)
    hs_ref[...] = h * dis


def _tc_final(accp_ref, hsp_ref, dis_ref, b_ref, batch_r_ref, batch_c_ref,
              gate_w_ref, gate_b_ref, lin_w_ref, lin_b_ref, out_ref):
    dis = dis_ref[...]
    x4 = dis * (_acc_rows(accp_ref) + hsp_ref[...]) + b_ref[...]
    x4 = jnp.maximum(x4, 0.0)
    gate = jnp.dot(x4, gate_w_ref[...],
                   preferred_element_type=jnp.float32) + gate_b_ref[...]
    gids_r = lax.broadcasted_iota(jnp.int32, (NNODE, NGRAPH), 1)
    oh = gids_r == batch_r_ref[...]
    masked = jnp.where(oh, gate, -1e30)
    m = jnp.max(masked, axis=0, keepdims=True)
    m = jnp.where(m > -1e29, m, 0.0)
    mb = jnp.sum(jnp.where(oh, m, 0.0), axis=1, keepdims=True)
    e = jnp.exp(gate - mb)
    ohf = oh.astype(jnp.float32)
    denom = jnp.sum(ohf * e, axis=0, keepdims=True)
    denom_b = jnp.sum(jnp.where(oh, denom, 0.0), axis=1, keepdims=True)
    alpha = e / jnp.maximum(denom_b, 1e-16)
    gids_c = lax.broadcasted_iota(jnp.int32, (NGRAPH, NNODE), 0)
    oht = (gids_c == batch_c_ref[...]).astype(jnp.float32)
    pooled = jnp.dot(oht, alpha * x4, preferred_element_type=jnp.float32)
    out_ref[...] = jnp.dot(pooled, lin_w_ref[...],
                           preferred_element_type=jnp.float32) + lin_b_ref[...]


def _tc_call(body, out_shape, *args):
    return pl.pallas_call(body, out_shape=out_shape)(*args)


def kernel(x, edge_index, batch, W0, b0, W1, b1, W2, b2, W3, b3,
           gate_w, gate_b, lin_w, lin_b):
    f32 = jnp.float32
    src = edge_index[0]
    dst = edge_index[1]
    pad = EPAD - NEDGE
    srcp = jnp.concatenate([src, jnp.zeros((pad,), jnp.int32)])
    dstp = jnp.concatenate([dst, jnp.full((pad,), NNODE, jnp.int32)])
    dstr = dstp.reshape(NW * NCHUNK, CH)
    pkr = ((srcp << PKSHIFT) | dstp).reshape(NW * NCHUNK, CH)
    ones16 = jnp.ones((CH, DW), f32)
    zeros16 = jnp.zeros((NPAD, DW), f32)

    degp = _deg_sc_built()(dstr, ones16, zeros16)
    degp2 = degp[:, :NNODE, :1]

    hs, dis = _tc_call(
        _tc_first,
        (jax.ShapeDtypeStruct((NNODE, FH), f32),
         jax.ShapeDtypeStruct((NNODE, 1), f32)),
        x, W0, degp2)

    for b, W in ((b0, W1), (b1, W2), (b2, W3)):
        accp = _agg_sc_built()(hs, pkr)
        hs = _tc_call(
            _tc_mid,
            jax.ShapeDtypeStruct((NNODE, FH), f32),
            accp[:, :NNODE, :], hs, dis, b.reshape(1, FH), W)

    accp = _agg_sc_built()(hs, pkr)
    out = _tc_call(
        _tc_final,
        jax.ShapeDtypeStruct((NGRAPH, NCLS), f32),
        accp[:, :NNODE, :], hs, dis, b3.reshape(1, FH),
        batch.reshape(NNODE, 1), batch.reshape(1, NNODE),
        gate_w, gate_b.reshape(1, 1), lin_w, lin_b.reshape(1, NCLS))
    return out


# speedup vs baseline: 1.0661x; 1.0520x over previous
import functools

import jax
import jax.numpy as jnp
from jax import lax
from jax.experimental import pallas as pl
from jax.experimental.pallas import tpu as pltpu
from jax.experimental.pallas import tpu_sc as plsc

NNODE = 10000
NEDGE = 320000
FIN = 128
FH = 64
NGRAPH = 64
NCLS = 10

NC = 2
NS = 16
NW = NC * NS

CH = 128
NCHUNK = 80
EPAD = NW * NCHUNK * CH
RPT = 632
NPAD = NS * RPT
DW = 16
KBUF = 4

PKSHIFT = 14
HS_RPT = 632


def _sc_mesh():
    return plsc.VectorSubcoreMesh(
        core_axis_name="c", subcore_axis_name="s", num_cores=NC, num_subcores=NS
    )


@functools.cache
def _deg_sc_built():
    return functools.partial(
        pl.kernel,
        out_type=jax.ShapeDtypeStruct((NC, NPAD, DW), jnp.float32),
        mesh=_sc_mesh(),
        scratch_types=[
            pltpu.VMEM((NCHUNK, CH), jnp.int32),
            pltpu.VMEM((CH, DW), jnp.float32),
            pltpu.VMEM_SHARED((NPAD, DW), jnp.float32),
            pltpu.SemaphoreType.DMA,
        ],
        compiler_params=pltpu.CompilerParams(use_tc_tiling_on_sc=False),
    )(_deg_sc_body)


def _deg_sc_body(dstr_hbm, ones_hbm, zeros16_hbm, out_hbm, dst_v, ones_v,
                 acc_sh, sem):
    c = lax.axis_index("c")
    s = lax.axis_index("s")
    wid = c * NS + s
    pltpu.sync_copy(dstr_hbm.at[pl.ds(wid * NCHUNK, NCHUNK)], dst_v)
    pltpu.sync_copy(ones_hbm, ones_v)
    pltpu.sync_copy(zeros16_hbm.at[pl.ds(s * RPT, RPT)], acc_sh.at[pl.ds(s * RPT, RPT)])
    plsc.subcore_barrier()

    def body(t, carry):
        base = t * KBUF
        for b in range(KBUF):
            pltpu.async_copy(ones_v, acc_sh.at[dst_v.at[base + b]], sem,
                             add=True)
        for b in range(KBUF):
            pltpu.make_async_copy(ones_v, acc_sh.at[dst_v.at[base + b]],
                                  sem).wait()
        return carry

    lax.fori_loop(0, NCHUNK // KBUF, body, 0)
    plsc.subcore_barrier()
    pltpu.sync_copy(acc_sh.at[pl.ds(s * RPT, RPT)], out_hbm.at[c, pl.ds(s * RPT, RPT)])


@functools.cache
def _agg_sc_built():
    return functools.partial(
        pl.kernel,
        out_type=jax.ShapeDtypeStruct((NC, NPAD, FH), jnp.float32),
        mesh=_sc_mesh(),
        scratch_types=[
            pltpu.VMEM((NCHUNK, CH), jnp.int32),
            pltpu.VMEM((KBUF, CH), jnp.int32),
            pltpu.VMEM((KBUF, CH), jnp.int32),
            [pltpu.VMEM((CH, FH), jnp.float32) for _ in range(KBUF)],
            [pltpu.SemaphoreType.DMA for _ in range(KBUF)],
            [pltpu.SemaphoreType.DMA for _ in range(KBUF)],
            pltpu.SemaphoreType.DMA,
            pltpu.VMEM_SHARED((NPAD, FH), jnp.float32),
            pltpu.VMEM_SHARED((NNODE, FH), jnp.float32),
        ],
        compiler_params=pltpu.CompilerParams(use_tc_tiling_on_sc=False),
    )(_agg_sc_body)


def _agg_sc_body(hs_hbm, pkr_hbm, out_hbm,
                 pk_v, src_v, dst_v, rows, gsem, ssem, hsem,
                 acc_sh, hs_sh):
    c = lax.axis_index("c")
    s = lax.axis_index("s")
    wid = c * NS + s

    @pl.when(s < NS - 1)
    def _():
        pltpu.async_copy(hs_hbm.at[pl.ds(s * HS_RPT, HS_RPT)],
                         hs_sh.at[pl.ds(s * HS_RPT, HS_RPT)], hsem)

    @pl.when(s == NS - 1)
    def _():
        pltpu.async_copy(
            hs_hbm.at[pl.ds((NS - 1) * HS_RPT, NNODE - (NS - 1) * HS_RPT)],
            hs_sh.at[pl.ds((NS - 1) * HS_RPT, NNODE - (NS - 1) * HS_RPT)],
            hsem)

    pltpu.sync_copy(pkr_hbm.at[pl.ds(wid * NCHUNK, NCHUNK)], pk_v)

    def _zrow(i, carry):
        for j in range(FH // 16):
            rows[0][i, pl.ds(j * 16, 16)] = jnp.zeros((16,), jnp.float32)
        return carry

    lax.fori_loop(0, CH, _zrow, 0)
    for k in range(RPT // CH):
        pltpu.sync_copy(rows[0], acc_sh.at[pl.ds(s * RPT + k * CH, CH)])
    rem = RPT % CH
    if rem:
        pltpu.sync_copy(rows[0].at[pl.ds(0, rem)],
                        acc_sh.at[pl.ds(s * RPT + (RPT // CH) * CH, rem)])

    @pl.when(s < NS - 1)
    def _():
        pltpu.make_async_copy(
            hs_hbm.at[pl.ds(s * HS_RPT, HS_RPT)],
            hs_sh.at[pl.ds(s * HS_RPT, HS_RPT)], hsem).wait()

    @pl.when(s == NS - 1)
    def _():
        pltpu.make_async_copy(
            hs_hbm.at[pl.ds((NS - 1) * HS_RPT, NNODE - (NS - 1) * HS_RPT)],
            hs_sh.at[pl.ds((NS - 1) * HS_RPT, NNODE - (NS - 1) * HS_RPT)],
            hsem).wait()

    plsc.subcore_barrier()

    def body(t, carry):
        base = t * KBUF
        for b in range(KBUF):
            for j in range(CH // 16):
                v = pk_v[base + b, pl.ds(j * 16, 16)]
                src_v[b, pl.ds(j * 16, 16)] = lax.shift_right_logical(
                    v, PKSHIFT)
                dst_v[b, pl.ds(j * 16, 16)] = v & ((1 << PKSHIFT) - 1)
        for b in range(KBUF):
            pltpu.async_copy(hs_sh.at[src_v.at[b]], rows[b], gsem[b])
        for b in range(KBUF):
            pltpu.make_async_copy(
                hs_sh.at[src_v.at[b]], rows[b], gsem[b]).wait()
            pltpu.async_copy(rows[b], acc_sh.at[dst_v.at[b]], ssem[b],
                             add=True)
        for b in range(KBUF):
            pltpu.make_async_copy(
                rows[b], acc_sh.at[dst_v.at[b]], ssem[b]).wait()
        return carry

    lax.fori_loop(0, NCHUNK // KBUF, body, 0)
    plsc.subcore_barrier()
    pltpu.sync_copy(acc_sh.at[pl.ds(s * RPT, RPT)],
                    out_hbm.at[c, pl.ds(s * RPT, RPT)])


def _tc_first(x_ref, w_ref, degp_ref, hs_ref, dis_ref):
    deg = degp_ref[0] + degp_ref[1] + 1.0
    dis = 1.0 / jnp.sqrt(deg)
    h = jnp.dot(x_ref[...], w_ref[...], preferred_element_type=jnp.float32)
    hs_ref[...] = h * dis
    dis_ref[...] = dis


def _acc_rows(accp_ref):
    return accp_ref[0] + accp_ref[1]


def _tc_mid(accp_ref, hsp_ref, dis_ref, b_ref, w_ref, hs_ref):
    dis = dis_ref[...]
    xk = dis * (_acc_rows(accp_ref) + hsp_ref[...]) + b_ref[...]
    xk = jnp.maximum(xk, 0.0)
    h = jnp.dot(xk, w_ref[...], preferred_element_type=jnp.float32)
    hs_ref[...] = h * dis


def _tc_final(accp_ref, hsp_ref, dis_ref, b_ref, batch_r_ref, batch_c_ref,
              gate_w_ref, gate_b_ref, lin_w_ref, lin_b_ref, out_ref):
    dis = dis_ref[...]
    x4 = dis * (_acc_rows(accp_ref) + hsp_ref[...]) + b_ref[...]
    x4 = jnp.maximum(x4, 0.0)
    gate = jnp.dot(x4, gate_w_ref[...],
                   preferred_element_type=jnp.float32) + gate_b_ref[...]
    gids_r = lax.broadcasted_iota(jnp.int32, (NNODE, NGRAPH), 1)
    oh = gids_r == batch_r_ref[...]
    masked = jnp.where(oh, gate, -1e30)
    m = jnp.max(masked, axis=0, keepdims=True)
    m = jnp.where(m > -1e29, m, 0.0)
    mb = jnp.sum(jnp.where(oh, m, 0.0), axis=1, keepdims=True)
    e = jnp.exp(gate - mb)
    ohf = oh.astype(jnp.float32)
    denom = jnp.sum(ohf * e, axis=0, keepdims=True)
    denom_b = jnp.sum(jnp.where(oh, denom, 0.0), axis=1, keepdims=True)
    alpha = e / jnp.maximum(denom_b, 1e-16)
    gids_c = lax.broadcasted_iota(jnp.int32, (NGRAPH, NNODE), 0)
    oht = (gids_c == batch_c_ref[...]).astype(jnp.float32)
    pooled = jnp.dot(oht, alpha * x4, preferred_element_type=jnp.float32)
    out_ref[...] = jnp.dot(pooled, lin_w_ref[...],
                           preferred_element_type=jnp.float32) + lin_b_ref[...]


def _tc_call(body, out_shape, *args):
    return pl.pallas_call(body, out_shape=out_shape)(*args)


def kernel(x, edge_index, batch, W0, b0, W1, b1, W2, b2, W3, b3,
           gate_w, gate_b, lin_w, lin_b):
    f32 = jnp.float32
    src = edge_index[0]
    dst = edge_index[1]
    pad = EPAD - NEDGE
    srcp = jnp.concatenate([src, jnp.zeros((pad,), jnp.int32)])
    dstp = jnp.concatenate([dst, jnp.full((pad,), NNODE, jnp.int32)])
    dstr = dstp.reshape(NW * NCHUNK, CH)
    pkr = ((srcp << PKSHIFT) | dstp).reshape(NW * NCHUNK, CH)
    ones16 = jnp.ones((CH, DW), f32)
    zeros16 = jnp.zeros((NPAD, DW), f32)

    degp = _deg_sc_built()(dstr, ones16, zeros16)
    degp2 = degp[:, :NNODE, :1]

    hs, dis = _tc_call(
        _tc_first,
        (jax.ShapeDtypeStruct((NNODE, FH), f32),
         jax.ShapeDtypeStruct((NNODE, 1), f32)),
        x, W0, degp2)

    for b, W in ((b0, W1), (b1, W2), (b2, W3)):
        accp = _agg_sc_built()(hs, pkr)
        hs = _tc_call(
            _tc_mid,
            jax.ShapeDtypeStruct((NNODE, FH), f32),
            accp[:, :NNODE, :], hs, dis, b.reshape(1, FH), W)

    accp = _agg_sc_built()(hs, pkr)
    out = _tc_call(
        _tc_final,
        jax.ShapeDtypeStruct((NGRAPH, NCLS), f32),
        accp[:, :NNODE, :], hs, dis, b3.reshape(1, FH),
        batch.reshape(NNODE, 1), batch.reshape(1, NNODE),
        gate_w, gate_b.reshape(1, 1), lin_w, lin_b.reshape(1, NCLS))
    return out
